# Initial kernel scaffold; baseline (speedup 1.0000x reference)
import jax
import jax.numpy as jnp
from jax.experimental import pallas as pl


def kernel(hidden_states, gate_weight, Wg, Wu, Wd):
    raise NotImplementedError("write your pallas kernel here")


# speedup vs baseline: 2.4368x; 2.4368x over previous
import functools

import jax
import jax.numpy as jnp
from jax.experimental import pallas as pl

TB = 256


def _moe_block(x_ref, gw_ref, wg_ref, wu_ref, wd_ref, o_ref):
    xb = x_ref[...]

    logits = jax.lax.dot_general(
        xb, gw_ref[...], (((1,), (1,)), ((), ())),
        preferred_element_type=jnp.float32)
    s = jax.nn.sigmoid(logits)
    E = s.shape[1]
    eidx = jax.lax.broadcasted_iota(jnp.int32, s.shape, 1)

    m1 = jnp.max(s, axis=1, keepdims=True)
    i1 = jnp.min(jnp.where(s == m1, eidx, E), axis=1, keepdims=True)
    s2 = jnp.where(eidx == i1, -jnp.inf, s)
    m2 = jnp.max(s2, axis=1, keepdims=True)
    i2 = jnp.min(jnp.where(s2 == m2, eidx, E), axis=1, keepdims=True)
    denom = m1 + m2 + 1e-20
    combine = (jnp.where(eidx == i1, m1, 0.0)
               + jnp.where(eidx == i2, m2, 0.0)) / denom

    xb16 = xb.astype(jnp.bfloat16)
    acc = jnp.zeros(xb.shape, jnp.float32)
    for e in range(E):
        g = jax.lax.dot_general(
            xb16, wg_ref[e], (((1,), (0,)), ((), ())),
            preferred_element_type=jnp.float32)
        u = jax.lax.dot_general(
            xb16, wu_ref[e], (((1,), (0,)), ((), ())),
            preferred_element_type=jnp.float32)
        h = (jax.nn.silu(g) * u).astype(jnp.bfloat16)
        o = jax.lax.dot_general(
            h, wd_ref[e], (((1,), (0,)), ((), ())),
            preferred_element_type=jnp.float32)
        acc = acc + combine[:, e:e + 1] * o
    o_ref[...] = acc


@jax.jit
def kernel(hidden_states, gate_weight, Wg, Wu, Wd):
    b, s, h = hidden_states.shape
    x = hidden_states.reshape(-1, h)
    T = x.shape[0]
    e = Wg.shape[0]
    f = Wg.shape[2]
    grid = (T // TB,)
    out = pl.pallas_call(
        _moe_block,
        grid=grid,
        in_specs=[
            pl.BlockSpec((TB, h), lambda i: (i, 0)),
            pl.BlockSpec((e, h), lambda i: (0, 0)),
            pl.BlockSpec((e, h, f), lambda i: (0, 0, 0)),
            pl.BlockSpec((e, h, f), lambda i: (0, 0, 0)),
            pl.BlockSpec((e, f, h), lambda i: (0, 0, 0)),
        ],
        out_specs=pl.BlockSpec((TB, h), lambda i: (i, 0)),
        out_shape=jax.ShapeDtypeStruct((T, h), jnp.float32),
    )(x, gate_weight,
      Wg.astype(jnp.bfloat16), Wu.astype(jnp.bfloat16),
      Wd.astype(jnp.bfloat16))
    return out.reshape(b, s, h)
